# Initial kernel scaffold; baseline (speedup 1.0000x reference)
import jax
import jax.numpy as jnp
from jax.experimental import pallas as pl


def kernel(x, edge_index, W_w, W_b, attention):
    raise NotImplementedError("write your pallas kernel here")


# speedup vs baseline: 41.8936x; 41.8936x over previous
import functools

import jax
import jax.numpy as jnp
from jax import lax
from jax.experimental import pallas as pl
from jax.experimental.pallas import tpu as pltpu
from jax.experimental.pallas import tpu_sc as plsc

N = 10000
E = 320000
DIN = 128
H = 4
HD = 32
FEAT = H * HD
ROW = FEAT + 16

NC = 2
NS = 16
NW = NC * NS
EPW = E // NW
C = 80
NCHUNK = EPW // C
N_ACC = 10240
RPS = N_ACC // NS

M_BLK = 400
GRID1 = N // M_BLK


def _tc_pre_body(x_ref, wt_ref, b_ref, p_ref, h_ref, s_ref, m_ref, mx):
    i = pl.program_id(0)
    h = jnp.dot(x_ref[...], wt_ref[...], preferred_element_type=jnp.float32)
    h = h + b_ref[...]
    s = jnp.dot(h, p_ref[...], preferred_element_type=jnp.float32)
    h_ref[...] = h
    s_ref[...] = s
    bm = jnp.max(s, axis=0, keepdims=True)

    @pl.when(i == 0)
    def _():
        mx[...] = bm

    @pl.when(i > 0)
    def _():
        mx[...] = jnp.maximum(mx[...], bm)

    t = mx[:, 0:16] + mx[:, 16:32]
    t = jnp.where(t > 0.0, t, 0.2 * t)
    lane = lax.broadcasted_iota(jnp.int32, (1, 16), 1)
    m_ref[...] = jnp.where(lane < H, t, 1e30)


def _tc_pre(x, wt, b, p):
    return pl.pallas_call(
        _tc_pre_body,
        grid=(GRID1,),
        in_specs=[
            pl.BlockSpec((M_BLK, DIN), lambda i: (i, 0)),
            pl.BlockSpec((DIN, FEAT), lambda i: (0, 0)),
            pl.BlockSpec((1, FEAT), lambda i: (0, 0)),
            pl.BlockSpec((DIN, 32), lambda i: (0, 0)),
        ],
        out_specs=[
            pl.BlockSpec((M_BLK, FEAT), lambda i: (i, 0)),
            pl.BlockSpec((M_BLK, 32), lambda i: (i, 0)),
            pl.BlockSpec((1, 16), lambda i: (0, 0)),
        ],
        out_shape=[
            jax.ShapeDtypeStruct((N, FEAT), jnp.float32),
            jax.ShapeDtypeStruct((N, 32), jnp.float32),
            jax.ShapeDtypeStruct((1, 16), jnp.float32),
        ],
        scratch_shapes=[pltpu.VMEM((1, 32), jnp.float32)],
    )(x, wt, b, p)


def _sc_body(h_hbm, s_hbm, m_hbm, src_hbm, dst_hbm, out_hbm,
             src_v, dst_v, s1r, s2r, hr, srow, m_v, acc,
             sem0, sem1, sem2):
    cid = lax.axis_index("c")
    sid = lax.axis_index("s")
    wid = sid * NC + cid

    zeros16 = jnp.zeros((16,), jnp.float32)

    @pl.loop(0, C)
    def _(r):
        for g in range(ROW // 16):
            srow[r, pl.ds(g * 16, 16)] = zeros16

    r0 = sid * RPS
    for t in range(RPS // C):
        pltpu.sync_copy(srow, acc.at[pl.ds(r0 + t * C, C)])
    plsc.subcore_barrier()

    pltpu.sync_copy(m_hbm, m_v)
    mvec = m_v[...]

    ebase = wid * EPW

    @pl.loop(0, NCHUNK)
    def _(k):
        base = ebase + k * C
        pltpu.sync_copy(src_hbm.at[pl.ds(base, C)], src_v)
        pltpu.sync_copy(dst_hbm.at[pl.ds(base, C)], dst_v)
        cp1 = pltpu.async_copy(s_hbm.at[src_v], s1r, sem0)
        cp2 = pltpu.async_copy(s_hbm.at[dst_v], s2r, sem1)
        cp3 = pltpu.async_copy(h_hbm.at[src_v], hr, sem2)
        cp1.wait()
        cp2.wait()
        cp3.wait()

        @pl.loop(0, C)
        def _(c):
            e = s1r[c, pl.ds(0, 16)] + s2r[c, pl.ds(16, 16)]
            e = jnp.where(e > 0.0, e, 0.2 * e)
            p = jnp.exp(e - mvec)
            srow[c, pl.ds(FEAT, 16)] = p
            cvec = jnp.full((16,), c, jnp.int32)
            for head in range(H):
                w = plsc.load_gather(
                    srow, [cvec, jnp.full((16,), FEAT + head, jnp.int32)])
                for g in range(HD // 16):
                    off = head * HD + g * 16
                    srow[c, pl.ds(off, 16)] = hr[c, pl.ds(off, 16)] * w

        pltpu.sync_copy(srow, acc.at[dst_v], add=True)

    plsc.subcore_barrier()
    pltpu.sync_copy(acc.at[pl.ds(r0, RPS)], out_hbm.at[cid, pl.ds(r0, RPS)])


def _sc_edge_pass(h, s, m, src, dst):
    mesh = plsc.VectorSubcoreMesh(core_axis_name="c", subcore_axis_name="s",
                                  num_cores=NC, num_subcores=NS)
    fn = pl.kernel(
        _sc_body,
        out_type=jax.ShapeDtypeStruct((NC, N_ACC, ROW), jnp.float32),
        mesh=mesh,
        scratch_types=[
            pltpu.VMEM((C,), jnp.int32),
            pltpu.VMEM((C,), jnp.int32),
            pltpu.VMEM((C, 32), jnp.float32),
            pltpu.VMEM((C, 32), jnp.float32),
            pltpu.VMEM((C, FEAT), jnp.float32),
            pltpu.VMEM((C, ROW), jnp.float32),
            pltpu.VMEM((16,), jnp.float32),
            pltpu.VMEM_SHARED((N_ACC, ROW), jnp.float32),
            pltpu.SemaphoreType.DMA,
            pltpu.SemaphoreType.DMA,
            pltpu.SemaphoreType.DMA,
        ],
        compiler_params=pltpu.CompilerParams(use_tc_tiling_on_sc=False,
                                             needs_layout_passes=False),
    )
    return fn(h, s, m, src, dst)


def _tc_post_body(part_ref, b_ref, out_ref):
    s = part_ref[0] + part_ref[1]
    num = s[:, 0:FEAT]
    den = jnp.dot(s[:, FEAT:ROW], b_ref[...],
                  preferred_element_type=jnp.float32)
    out_ref[...] = num / jnp.maximum(den, 1e-16)


def _tc_post(part, bmat):
    return pl.pallas_call(
        _tc_post_body,
        grid=(GRID1,),
        in_specs=[
            pl.BlockSpec((NC, M_BLK, ROW), lambda i: (0, i, 0)),
            pl.BlockSpec((16, FEAT), lambda i: (0, 0)),
        ],
        out_specs=pl.BlockSpec((M_BLK, FEAT), lambda i: (i, 0)),
        out_shape=jax.ShapeDtypeStruct((N, FEAT), jnp.float32),
    )(part, bmat)


def kernel(x, edge_index, W_w, W_b, attention):
    a1 = attention[:HD].reshape(HD, 1)
    a2 = attention[HD:].reshape(HD, 1)
    eye = jnp.eye(H, dtype=jnp.float32)
    p1 = jnp.pad(jnp.kron(eye, a1), ((0, 0), (0, 16 - H)))
    p2 = jnp.pad(jnp.kron(eye, a2), ((0, 0), (0, 16 - H)))
    pmat = jnp.concatenate([p1, p2], axis=1)
    bmat = jnp.pad(jnp.kron(eye, jnp.ones((1, HD), jnp.float32)),
                   ((0, 16 - H), (0, 0)))

    h, s, m = _tc_pre(x, W_w.T, W_b.reshape(1, FEAT), pmat)
    part = _sc_edge_pass(h, s, m.reshape(16), edge_index[0], edge_index[1])
    return _tc_post(part, bmat)


# speedup vs baseline: 63.2443x; 1.5096x over previous
import jax
import jax.numpy as jnp
from jax import lax
from jax.experimental import pallas as pl
from jax.experimental.pallas import tpu as pltpu
from jax.experimental.pallas import tpu_sc as plsc

N = 10000
E = 320000
DIN = 128
H = 4
HD = 32
FEAT = H * HD
ROW = FEAT + 16

NC = 2
NS = 16
NW = NC * NS
EPW = E // NW
C = 80
NCHUNK = EPW // C
N_ACC = 10240
RPS = N_ACC // NS

M_BLK = 400
GRID1 = N // M_BLK


def _tc_pre_body(x_ref, wt_ref, b_ref, p_ref, hx_ref, s2_ref, m_ref, mx):
    i = pl.program_id(0)
    h = jnp.dot(x_ref[...], wt_ref[...], preferred_element_type=jnp.float32)
    h = h + b_ref[...]
    s = jnp.dot(h, p_ref[...], preferred_element_type=jnp.float32)
    hx_ref[:, 0:FEAT] = h
    hx_ref[:, FEAT:ROW] = s[:, 0:16]
    s2_ref[...] = s[:, 16:32]
    bm = jnp.max(s, axis=0, keepdims=True)

    @pl.when(i == 0)
    def _():
        mx[...] = bm

    @pl.when(i > 0)
    def _():
        mx[...] = jnp.maximum(mx[...], bm)

    t = mx[:, 0:16] + mx[:, 16:32]
    t = jnp.where(t > 0.0, t, 0.2 * t)
    lane = lax.broadcasted_iota(jnp.int32, (1, 16), 1)
    m_ref[...] = jnp.where(lane < H, t, 1e30)


def _tc_pre(x, wt, b, p):
    return pl.pallas_call(
        _tc_pre_body,
        grid=(GRID1,),
        in_specs=[
            pl.BlockSpec((M_BLK, DIN), lambda i: (i, 0)),
            pl.BlockSpec((DIN, FEAT), lambda i: (0, 0)),
            pl.BlockSpec((1, FEAT), lambda i: (0, 0)),
            pl.BlockSpec((DIN, 32), lambda i: (0, 0)),
        ],
        out_specs=[
            pl.BlockSpec((M_BLK, ROW), lambda i: (i, 0)),
            pl.BlockSpec((M_BLK, 16), lambda i: (i, 0)),
            pl.BlockSpec((1, 16), lambda i: (0, 0)),
        ],
        out_shape=[
            jax.ShapeDtypeStruct((N, ROW), jnp.float32),
            jax.ShapeDtypeStruct((N, 16), jnp.float32),
            jax.ShapeDtypeStruct((1, 16), jnp.float32),
        ],
        scratch_shapes=[pltpu.VMEM((1, 32), jnp.float32)],
    )(x, wt, b, p)


def _sc_body(hx_hbm, s2_hbm, m_hbm, src_hbm, dst_hbm, out_hbm,
             src_v, dst_v, hxr, s2r, m_v, acc, sem0, sem1):
    cid = lax.axis_index("c")
    sid = lax.axis_index("s")
    wid = sid * NC + cid

    zeros16 = jnp.zeros((16,), jnp.float32)

    @pl.loop(0, C)
    def _(r):
        for g in range(ROW // 16):
            hxr[r, pl.ds(g * 16, 16)] = zeros16

    r0 = sid * RPS
    for t in range(RPS // C):
        pltpu.sync_copy(hxr, acc.at[pl.ds(r0 + t * C, C)])
    plsc.subcore_barrier()

    pltpu.sync_copy(m_hbm, m_v)
    mvec = m_v[...]
    gdn = lax.GatherDimensionNumbers(
        offset_dims=(), collapsed_slice_dims=(0,), start_index_map=(0,))
    bidx = [jnp.full((16, 1), head, jnp.int32) for head in range(H)]

    ebase = wid * EPW

    @pl.loop(0, NCHUNK)
    def _(k):
        base = ebase + k * C
        pltpu.sync_copy(src_hbm.at[pl.ds(base, C)], src_v)
        pltpu.sync_copy(dst_hbm.at[pl.ds(base, C)], dst_v)
        cp1 = pltpu.async_copy(hx_hbm.at[src_v], hxr, sem0)
        cp2 = pltpu.async_copy(s2_hbm.at[dst_v], s2r, sem1)
        cp1.wait()
        cp2.wait()

        @pl.loop(0, C)
        def _(c):
            e = hxr[c, pl.ds(FEAT, 16)] + s2r[c, pl.ds(0, 16)]
            e = jnp.where(e > 0.0, e, 0.2 * e)
            p = jnp.exp(e - mvec)
            hxr[c, pl.ds(FEAT, 16)] = p
            for head in range(H):
                w = lax.gather(
                    p, bidx[head], dimension_numbers=gdn, slice_sizes=(1,),
                    mode=lax.GatherScatterMode.PROMISE_IN_BOUNDS)
                for g in range(HD // 16):
                    off = head * HD + g * 16
                    hxr[c, pl.ds(off, 16)] = hxr[c, pl.ds(off, 16)] * w

        pltpu.sync_copy(hxr, acc.at[dst_v], add=True)

    plsc.subcore_barrier()
    pltpu.sync_copy(acc.at[pl.ds(r0, RPS)], out_hbm.at[cid, pl.ds(r0, RPS)])


def _sc_edge_pass(hx, s2, m, src, dst):
    mesh = plsc.VectorSubcoreMesh(core_axis_name="c", subcore_axis_name="s",
                                  num_cores=NC, num_subcores=NS)
    fn = pl.kernel(
        _sc_body,
        out_type=jax.ShapeDtypeStruct((NC, N_ACC, ROW), jnp.float32),
        mesh=mesh,
        scratch_types=[
            pltpu.VMEM((C,), jnp.int32),
            pltpu.VMEM((C,), jnp.int32),
            pltpu.VMEM((C, ROW), jnp.float32),
            pltpu.VMEM((C, 16), jnp.float32),
            pltpu.VMEM((16,), jnp.float32),
            pltpu.VMEM_SHARED((N_ACC, ROW), jnp.float32),
            pltpu.SemaphoreType.DMA,
            pltpu.SemaphoreType.DMA,
        ],
        compiler_params=pltpu.CompilerParams(use_tc_tiling_on_sc=False,
                                             needs_layout_passes=False),
    )
    return fn(hx, s2, m, src, dst)


def _tc_post_body(part_ref, b_ref, out_ref):
    s = part_ref[0] + part_ref[1]
    num = s[:, 0:FEAT]
    den = jnp.dot(s[:, FEAT:ROW], b_ref[...],
                  preferred_element_type=jnp.float32)
    out_ref[...] = num / jnp.maximum(den, 1e-16)


def _tc_post(part, bmat):
    return pl.pallas_call(
        _tc_post_body,
        grid=(GRID1,),
        in_specs=[
            pl.BlockSpec((NC, M_BLK, ROW), lambda i: (0, i, 0)),
            pl.BlockSpec((16, FEAT), lambda i: (0, 0)),
        ],
        out_specs=pl.BlockSpec((M_BLK, FEAT), lambda i: (i, 0)),
        out_shape=jax.ShapeDtypeStruct((N, FEAT), jnp.float32),
    )(part, bmat)


def kernel(x, edge_index, W_w, W_b, attention):
    a1 = attention[:HD].reshape(HD, 1)
    a2 = attention[HD:].reshape(HD, 1)
    eye = jnp.eye(H, dtype=jnp.float32)
    p1 = jnp.pad(jnp.kron(eye, a1), ((0, 0), (0, 16 - H)))
    p2 = jnp.pad(jnp.kron(eye, a2), ((0, 0), (0, 16 - H)))
    pmat = jnp.concatenate([p1, p2], axis=1)
    bmat = jnp.pad(jnp.kron(eye, jnp.ones((1, HD), jnp.float32)),
                   ((0, 16 - H), (0, 0)))

    hx, s2, m = _tc_pre(x, W_w.T, W_b.reshape(1, FEAT), pmat)
    part = _sc_edge_pass(hx, s2, m.reshape(16), edge_index[0], edge_index[1])
    return _tc_post(part, bmat)


# speedup vs baseline: 90.0332x; 1.4236x over previous
import jax
import jax.numpy as jnp
from jax import lax
from jax.experimental import pallas as pl
from jax.experimental.pallas import tpu as pltpu
from jax.experimental.pallas import tpu_sc as plsc

N = 10000
E = 320000
DIN = 128
H = 4
HD = 32
FEAT = H * HD
ROW = FEAT + 16

NC = 2
NS = 16
NW = NC * NS
EPW = E // NW
C = 80
NCHUNK = EPW // C
N_ACC = 10240
RPS = N_ACC // NS

M_BLK = 400
GRID1 = N // M_BLK


def _tc_pre_body(x_ref, wt_ref, b_ref, p_ref, hx_ref, s2_ref, m_ref, mx):
    i = pl.program_id(0)
    h = jnp.dot(x_ref[...], wt_ref[...], preferred_element_type=jnp.float32)
    h = h + b_ref[...]
    s = jnp.dot(h, p_ref[...], preferred_element_type=jnp.float32)
    hx_ref[:, 0:FEAT] = h
    hx_ref[:, FEAT:ROW] = s[:, 0:16]
    s2_ref[...] = s[:, 16:32]
    bm = jnp.max(s, axis=0, keepdims=True)

    @pl.when(i == 0)
    def _():
        mx[...] = bm

    @pl.when(i > 0)
    def _():
        mx[...] = jnp.maximum(mx[...], bm)

    t = mx[:, 0:16] + mx[:, 16:32]
    t = jnp.where(t > 0.0, t, 0.2 * t)
    lane = lax.broadcasted_iota(jnp.int32, (1, 16), 1)
    m_ref[...] = jnp.where(lane < H, t, 1e30)


def _tc_pre(x, wt, b, p):
    return pl.pallas_call(
        _tc_pre_body,
        grid=(GRID1,),
        in_specs=[
            pl.BlockSpec((M_BLK, DIN), lambda i: (i, 0)),
            pl.BlockSpec((DIN, FEAT), lambda i: (0, 0)),
            pl.BlockSpec((1, FEAT), lambda i: (0, 0)),
            pl.BlockSpec((DIN, 32), lambda i: (0, 0)),
        ],
        out_specs=[
            pl.BlockSpec((M_BLK, ROW), lambda i: (i, 0)),
            pl.BlockSpec((M_BLK, 16), lambda i: (i, 0)),
            pl.BlockSpec((1, 16), lambda i: (0, 0)),
        ],
        out_shape=[
            jax.ShapeDtypeStruct((N, ROW), jnp.float32),
            jax.ShapeDtypeStruct((N, 16), jnp.float32),
            jax.ShapeDtypeStruct((1, 16), jnp.float32),
        ],
        scratch_shapes=[pltpu.VMEM((1, 32), jnp.float32)],
    )(x, wt, b, p)


NBUF = 3


def _sc_body(hx_hbm, s2_hbm, m_hbm, src_hbm, dst_hbm, out_hbm,
             src_v, dst_v, hxr, s2r, m_v, acc, gx, gs, ss):
    cid = lax.axis_index("c")
    sid = lax.axis_index("s")
    wid = sid * NC + cid

    zeros16 = jnp.zeros((16,), jnp.float32)

    @pl.loop(0, C)
    def _(r):
        for g in range(ROW // 16):
            hxr[0][r, pl.ds(g * 16, 16)] = zeros16

    r0 = sid * RPS
    for t in range(RPS // C):
        pltpu.sync_copy(hxr[0], acc.at[pl.ds(r0 + t * C, C)])
    plsc.subcore_barrier()

    pltpu.sync_copy(m_hbm, m_v)
    mvec = m_v[...]
    gdn = lax.GatherDimensionNumbers(
        offset_dims=(), collapsed_slice_dims=(0,), start_index_map=(0,))
    bidx = [jnp.full((16, 1), head, jnp.int32) for head in range(H)]

    ebase = wid * EPW

    def issue(k, b):
        base = ebase + k * C
        pltpu.sync_copy(src_hbm.at[pl.ds(base, C)], src_v[b])
        pltpu.sync_copy(dst_hbm.at[pl.ds(base, C)], dst_v[b])
        pltpu.async_copy(hx_hbm.at[src_v[b]], hxr[b], gx[b])
        pltpu.async_copy(s2_hbm.at[dst_v[b]], s2r[b], gs[b])

    def wait_gathers(b):
        pltpu.make_async_copy(hx_hbm.at[src_v[b]], hxr[b], gx[b]).wait()
        pltpu.make_async_copy(s2_hbm.at[dst_v[b]], s2r[b], gs[b]).wait()

    def compute(b):
        @pl.loop(0, C, unroll=2)
        def _(c):
            e = hxr[b][c, pl.ds(FEAT, 16)] + s2r[b][c, pl.ds(0, 16)]
            e = jnp.where(e > 0.0, e, 0.2 * e)
            p = jnp.exp(e - mvec)
            hxr[b][c, pl.ds(FEAT, 16)] = p
            for head in range(H):
                w = lax.gather(
                    p, bidx[head], dimension_numbers=gdn, slice_sizes=(1,),
                    mode=lax.GatherScatterMode.PROMISE_IN_BOUNDS)
                for g in range(HD // 16):
                    off = head * HD + g * 16
                    hxr[b][c, pl.ds(off, 16)] = hxr[b][c, pl.ds(off, 16)] * w

    def start_scatter(b):
        pltpu.async_copy(hxr[b], acc.at[dst_v[b]], ss[b], add=True)

    def wait_scatter(b):
        pltpu.make_async_copy(hxr[b], acc.at[dst_v[b]], ss[b]).wait()

    issue(0, 2)
    wait_gathers(2)
    compute(2)
    start_scatter(2)
    wait_scatter(2)
    issue(1, 2)
    wait_gathers(2)
    compute(2)
    start_scatter(2)
    issue(2, 0)
    issue(3, 1)

    @pl.loop(0, (NCHUNK - 2) // NBUF)
    def _(t):
        for p in range(NBUF):
            k = 2 + t * NBUF + p
            brefill = (p + 2) % NBUF
            wait_gathers(p)
            compute(p)
            start_scatter(p)
            wait_scatter(brefill)
            krefill = k + 2

            @pl.when(krefill < NCHUNK)
            def _():
                issue(krefill, brefill)

    wait_scatter(2)
    plsc.subcore_barrier()
    pltpu.sync_copy(acc.at[pl.ds(r0, RPS)], out_hbm.at[cid, pl.ds(r0, RPS)])


def _sc_edge_pass(hx, s2, m, src, dst):
    mesh = plsc.VectorSubcoreMesh(core_axis_name="c", subcore_axis_name="s",
                                  num_cores=NC, num_subcores=NS)
    fn = pl.kernel(
        _sc_body,
        out_type=jax.ShapeDtypeStruct((NC, N_ACC, ROW), jnp.float32),
        mesh=mesh,
        scratch_types=[
            [pltpu.VMEM((C,), jnp.int32)] * NBUF,
            [pltpu.VMEM((C,), jnp.int32)] * NBUF,
            [pltpu.VMEM((C, ROW), jnp.float32)] * NBUF,
            [pltpu.VMEM((C, 16), jnp.float32)] * NBUF,
            pltpu.VMEM((16,), jnp.float32),
            pltpu.VMEM_SHARED((N_ACC, ROW), jnp.float32),
            [pltpu.SemaphoreType.DMA] * NBUF,
            [pltpu.SemaphoreType.DMA] * NBUF,
            [pltpu.SemaphoreType.DMA] * NBUF,
        ],
        compiler_params=pltpu.CompilerParams(use_tc_tiling_on_sc=False,
                                             needs_layout_passes=False),
    )
    return fn(hx, s2, m, src, dst)


def _tc_post_body(part_ref, b_ref, out_ref):
    s = part_ref[0] + part_ref[1]
    num = s[:, 0:FEAT]
    den = jnp.dot(s[:, FEAT:ROW], b_ref[...],
                  preferred_element_type=jnp.float32)
    out_ref[...] = num / jnp.maximum(den, 1e-16)


def _tc_post(part, bmat):
    return pl.pallas_call(
        _tc_post_body,
        grid=(GRID1,),
        in_specs=[
            pl.BlockSpec((NC, M_BLK, ROW), lambda i: (0, i, 0)),
            pl.BlockSpec((16, FEAT), lambda i: (0, 0)),
        ],
        out_specs=pl.BlockSpec((M_BLK, FEAT), lambda i: (i, 0)),
        out_shape=jax.ShapeDtypeStruct((N, FEAT), jnp.float32),
    )(part, bmat)


def kernel(x, edge_index, W_w, W_b, attention):
    a1 = attention[:HD].reshape(HD, 1)
    a2 = attention[HD:].reshape(HD, 1)
    eye = jnp.eye(H, dtype=jnp.float32)
    p1 = jnp.pad(jnp.kron(eye, a1), ((0, 0), (0, 16 - H)))
    p2 = jnp.pad(jnp.kron(eye, a2), ((0, 0), (0, 16 - H)))
    pmat = jnp.concatenate([p1, p2], axis=1)
    bmat = jnp.pad(jnp.kron(eye, jnp.ones((1, HD), jnp.float32)),
                   ((0, 16 - H), (0, 0)))

    hx, s2, m = _tc_pre(x, W_w.T, W_b.reshape(1, FEAT), pmat)
    part = _sc_edge_pass(hx, s2, m.reshape(16), edge_index[0], edge_index[1])
    return _tc_post(part, bmat)


# speedup vs baseline: 90.3159x; 1.0031x over previous
import jax
import jax.numpy as jnp
from jax import lax
from jax.experimental import pallas as pl
from jax.experimental.pallas import tpu as pltpu
from jax.experimental.pallas import tpu_sc as plsc

N = 10000
E = 320000
DIN = 128
H = 4
HD = 32
FEAT = H * HD
ROW = FEAT + 16

NC = 2
NS = 16
NW = NC * NS
EPW = E // NW
C = 80
NCHUNK = EPW // C
N_ACC = 10240
RPS = N_ACC // NS

M_BLK = 400
GRID1 = N // M_BLK


def _tc_pre_body(x_ref, wt_ref, b_ref, p_ref, hx_ref, s2_ref, m_ref, mx):
    i = pl.program_id(0)
    h = jnp.dot(x_ref[...], wt_ref[...], preferred_element_type=jnp.float32)
    h = h + b_ref[...]
    s = jnp.dot(h, p_ref[...], preferred_element_type=jnp.float32)
    hx_ref[:, 0:FEAT] = h
    hx_ref[:, FEAT:ROW] = s[:, 0:16]
    s2_ref[...] = s[:, 16:32]
    bm = jnp.max(s, axis=0, keepdims=True)

    @pl.when(i == 0)
    def _():
        mx[...] = bm

    @pl.when(i > 0)
    def _():
        mx[...] = jnp.maximum(mx[...], bm)

    t = mx[:, 0:16] + mx[:, 16:32]
    t = jnp.where(t > 0.0, t, 0.2 * t)
    lane = lax.broadcasted_iota(jnp.int32, (1, 16), 1)
    m_ref[...] = jnp.where(lane < H, t, 1e30)


def _tc_pre(x, wt, b, p):
    return pl.pallas_call(
        _tc_pre_body,
        grid=(GRID1,),
        in_specs=[
            pl.BlockSpec((M_BLK, DIN), lambda i: (i, 0)),
            pl.BlockSpec((DIN, FEAT), lambda i: (0, 0)),
            pl.BlockSpec((1, FEAT), lambda i: (0, 0)),
            pl.BlockSpec((DIN, 32), lambda i: (0, 0)),
        ],
        out_specs=[
            pl.BlockSpec((M_BLK, ROW), lambda i: (i, 0)),
            pl.BlockSpec((M_BLK, 16), lambda i: (i, 0)),
            pl.BlockSpec((1, 16), lambda i: (0, 0)),
        ],
        out_shape=[
            jax.ShapeDtypeStruct((N, ROW), jnp.float32),
            jax.ShapeDtypeStruct((N, 16), jnp.float32),
            jax.ShapeDtypeStruct((1, 16), jnp.float32),
        ],
        scratch_shapes=[pltpu.VMEM((1, 32), jnp.float32)],
    )(x, wt, b, p)


NBUF = 3


def _sc_body(hx_hbm, s2_hbm, m_hbm, src_hbm, dst_hbm, out_hbm,
             srcs, dsts, hxr, s2r, m_v, acc, gx, gs, ss):
    cid = lax.axis_index("c")
    sid = lax.axis_index("s")
    wid = sid * NC + cid

    zeros16 = jnp.zeros((16,), jnp.float32)

    @pl.loop(0, C)
    def _(r):
        for g in range(ROW // 16):
            hxr[0][r, pl.ds(g * 16, 16)] = zeros16

    r0 = sid * RPS
    for t in range(RPS // C):
        pltpu.sync_copy(hxr[0], acc.at[pl.ds(r0 + t * C, C)])
    plsc.subcore_barrier()

    pltpu.sync_copy(m_hbm, m_v)
    ebase = wid * EPW
    mvec = m_v[...]
    gdn = lax.GatherDimensionNumbers(
        offset_dims=(), collapsed_slice_dims=(0,), start_index_map=(0,))
    bidx = [jnp.full((16, 1), head, jnp.int32) for head in range(H)]

    def issue(k, b):
        base = ebase + k * C
        pltpu.sync_copy(src_hbm.at[pl.ds(base, C)], srcs[b])
        pltpu.sync_copy(dst_hbm.at[pl.ds(base, C)], dsts[b])
        pltpu.async_copy(hx_hbm.at[srcs[b]], hxr[b], gx[b])
        pltpu.async_copy(s2_hbm.at[dsts[b]], s2r[b], gs[b])

    def wait_gathers(b):
        pltpu.make_async_copy(hx_hbm.at[srcs[b]], hxr[b], gx[b]).wait()
        pltpu.make_async_copy(s2_hbm.at[dsts[b]], s2r[b], gs[b]).wait()

    def compute(b):
        @pl.loop(0, C, unroll=4)
        def _(c):
            e = hxr[b][c, pl.ds(FEAT, 16)] + s2r[b][c, pl.ds(0, 16)]
            e = jnp.where(e > 0.0, e, 0.2 * e)
            p = jnp.exp(e - mvec)
            hxr[b][c, pl.ds(FEAT, 16)] = p
            for head in range(H):
                w = lax.gather(
                    p, bidx[head], dimension_numbers=gdn, slice_sizes=(1,),
                    mode=lax.GatherScatterMode.PROMISE_IN_BOUNDS)
                for g in range(HD // 16):
                    off = head * HD + g * 16
                    hxr[b][c, pl.ds(off, 16)] = hxr[b][c, pl.ds(off, 16)] * w

    def start_scatter(k, b):
        pltpu.async_copy(hxr[b], acc.at[dsts[b]], ss[b], add=True)

    def wait_scatter(b):
        pltpu.make_async_copy(hxr[b], acc.at[dsts[0]], ss[b]).wait()

    issue(0, 2)
    wait_gathers(2)
    compute(2)
    start_scatter(0, 2)
    wait_scatter(2)
    issue(1, 2)
    wait_gathers(2)
    compute(2)
    start_scatter(1, 2)
    issue(2, 0)
    issue(3, 1)

    @pl.loop(0, (NCHUNK - 2) // NBUF)
    def _(t):
        for p in range(NBUF):
            k = 2 + t * NBUF + p
            brefill = (p + 2) % NBUF
            wait_gathers(p)
            compute(p)
            start_scatter(k, p)
            wait_scatter(brefill)
            krefill = k + 2

            @pl.when(krefill < NCHUNK)
            def _():
                issue(krefill, brefill)

    wait_scatter(2)
    plsc.subcore_barrier()
    pltpu.sync_copy(acc.at[pl.ds(r0, RPS)], out_hbm.at[cid, pl.ds(r0, RPS)])


def _sc_edge_pass(hx, s2, m, src, dst):
    mesh = plsc.VectorSubcoreMesh(core_axis_name="c", subcore_axis_name="s",
                                  num_cores=NC, num_subcores=NS)
    fn = pl.kernel(
        _sc_body,
        out_type=jax.ShapeDtypeStruct((NC, N_ACC, ROW), jnp.float32),
        mesh=mesh,
        scratch_types=[
            [pltpu.VMEM((C,), jnp.int32)] * NBUF,
            [pltpu.VMEM((C,), jnp.int32)] * NBUF,
            [pltpu.VMEM((C, ROW), jnp.float32)] * NBUF,
            [pltpu.VMEM((C, 16), jnp.float32)] * NBUF,
            pltpu.VMEM((16,), jnp.float32),
            pltpu.VMEM_SHARED((N_ACC, ROW), jnp.float32),
            [pltpu.SemaphoreType.DMA] * NBUF,
            [pltpu.SemaphoreType.DMA] * NBUF,
            [pltpu.SemaphoreType.DMA] * NBUF,
        ],
        compiler_params=pltpu.CompilerParams(use_tc_tiling_on_sc=False,
                                             needs_layout_passes=False),
    )
    return fn(hx, s2, m, src, dst)


def _tc_post_body(part_ref, b_ref, out_ref):
    s = part_ref[0] + part_ref[1]
    num = s[:, 0:FEAT]
    den = jnp.dot(s[:, FEAT:ROW], b_ref[...],
                  preferred_element_type=jnp.float32)
    out_ref[...] = num / jnp.maximum(den, 1e-16)


def _tc_post(part, bmat):
    return pl.pallas_call(
        _tc_post_body,
        grid=(GRID1,),
        in_specs=[
            pl.BlockSpec((NC, M_BLK, ROW), lambda i: (0, i, 0)),
            pl.BlockSpec((16, FEAT), lambda i: (0, 0)),
        ],
        out_specs=pl.BlockSpec((M_BLK, FEAT), lambda i: (i, 0)),
        out_shape=jax.ShapeDtypeStruct((N, FEAT), jnp.float32),
    )(part, bmat)


def kernel(x, edge_index, W_w, W_b, attention):
    a1 = attention[:HD].reshape(HD, 1)
    a2 = attention[HD:].reshape(HD, 1)
    eye = jnp.eye(H, dtype=jnp.float32)
    p1 = jnp.pad(jnp.kron(eye, a1), ((0, 0), (0, 16 - H)))
    p2 = jnp.pad(jnp.kron(eye, a2), ((0, 0), (0, 16 - H)))
    pmat = jnp.concatenate([p1, p2], axis=1)
    bmat = jnp.pad(jnp.kron(eye, jnp.ones((1, HD), jnp.float32)),
                   ((0, 16 - H), (0, 0)))

    hx, s2, m = _tc_pre(x, W_w.T, W_b.reshape(1, FEAT), pmat)
    part = _sc_edge_pass(hx, s2, m.reshape(16), edge_index[0], edge_index[1])
    return _tc_post(part, bmat)
